# baseline (speedup 1.0000x reference)
import functools

import jax
import jax.numpy as jnp
from jax import lax
from jax.experimental import pallas as pl
from jax.experimental.pallas import tpu as pltpu
from jax.experimental.pallas import tpu_sc as plsc

N = 10000
E = 320000
D = 128

NC = 2
NS = 16
NW = NC * NS
CH = 80
EPW = E // NW
NITER = EPW // CH
NBC = 25
NBLK = NITER // NBC
N_PAD = 10240
RPS = N_PAD // NS
ZB = 32


def _make_sc_agg(with_deg: bool):
    mesh = plsc.VectorSubcoreMesh(core_axis_name="c", subcore_axis_name="s")
    out_type = [jax.ShapeDtypeStruct((NC, N_PAD, D), jnp.float32)]
    if with_deg:
        out_type.append(jax.ShapeDtypeStruct((NC, N_PAD, 16), jnp.float32))
    scratch = [
        pltpu.VMEM((2, NBC, CH), jnp.int32),
        pltpu.VMEM((2, NBC, CH), jnp.int32),
        pltpu.VMEM((2, CH, D), jnp.float32),
        pltpu.VMEM((ZB, D), jnp.float32),
        pltpu.VMEM_SHARED((N_PAD, D), jnp.float32),
        pltpu.SemaphoreType.DMA,
        pltpu.SemaphoreType.DMA,
        pltpu.SemaphoreType.DMA,
    ]
    if with_deg:
        scratch += [
            pltpu.VMEM((CH, 16), jnp.float32),
            pltpu.VMEM((ZB, 16), jnp.float32),
            pltpu.VMEM_SHARED((N_PAD, 16), jnp.float32),
        ]

    @functools.partial(pl.kernel, out_type=tuple(out_type), mesh=mesh,
                       scratch_types=scratch,
                       compiler_params=pltpu.CompilerParams(
                           use_tc_tiling_on_sc=False))
    def sc_kernel(src_hbm, dst_hbm, x_hbm, *refs):
        if with_deg:
            (agg_out, deg_out, src_b, dst_b, rows_v, zrow_v, agg_sh,
             semg, semi, semz, ones_v, zdeg_v, deg_sh) = refs
        else:
            (agg_out, src_b, dst_b, rows_v, zrow_v, agg_sh,
             semg, semi, semz) = refs
        c = lax.axis_index("c")
        s = lax.axis_index("s")
        wid = s * NC + c
        zero16 = jnp.zeros((16,), jnp.float32)

        pltpu.sync_copy(src_hbm.at[wid, pl.ds(0, NBC)], src_b.at[0])
        pltpu.sync_copy(dst_hbm.at[wid, pl.ds(0, NBC)], dst_b.at[0])
        pltpu.async_copy(src_hbm.at[wid, pl.ds(NBC, NBC)], src_b.at[1], semi)
        pltpu.async_copy(dst_hbm.at[wid, pl.ds(NBC, NBC)], dst_b.at[1], semi)

        @pl.loop(0, ZB)
        def _(i):
            @pl.loop(0, D, step=16)
            def _(j):
                zrow_v[i, pl.ds(j, 16)] = zero16

        if with_deg:
            @pl.loop(0, ZB)
            def _(i):
                zdeg_v[i, pl.ds(0, 16)] = zero16

            @pl.loop(0, CH)
            def _(i):
                ones_v[i, pl.ds(0, 16)] = jnp.ones((16,), jnp.float32)

        @pl.loop(0, RPS, step=ZB)
        def _(k):
            pltpu.async_copy(zrow_v, agg_sh.at[pl.ds(s * RPS + k, ZB)], semz)
        if with_deg:
            @pl.loop(0, RPS, step=ZB)
            def _(k):
                pltpu.async_copy(zdeg_v, deg_sh.at[pl.ds(s * RPS + k, ZB)],
                                 semz)

        @pl.loop(0, RPS, step=ZB)
        def _(k):
            pltpu.make_async_copy(
                zrow_v, agg_sh.at[pl.ds(s * RPS + k, ZB)], semz).wait()
        if with_deg:
            @pl.loop(0, RPS, step=ZB)
            def _(k):
                pltpu.make_async_copy(
                    zdeg_v, deg_sh.at[pl.ds(s * RPS + k, ZB)], semz).wait()
        plsc.subcore_barrier()

        pltpu.async_copy(x_hbm.at[src_b.at[0, 0]], rows_v.at[0], semg)

        @pl.loop(0, NITER)
        def _(i):
            p = lax.rem(i, 2)
            q = 1 - p
            lb = lax.div(i, NBC)
            j = lax.rem(i, NBC)
            m = lax.rem(lb, 2)
            i1 = i + 1
            lb1 = lax.div(i1, NBC)
            j1 = lax.rem(i1, NBC)
            m1 = lax.rem(lb1, 2)

            @pl.when(i1 < NITER)
            def _():
                @pl.when(j1 == 0)
                def _():
                    pltpu.make_async_copy(
                        src_hbm.at[wid, pl.ds(lb1 * NBC, NBC)],
                        src_b.at[m1], semi).wait()
                    pltpu.make_async_copy(
                        dst_hbm.at[wid, pl.ds(lb1 * NBC, NBC)],
                        dst_b.at[m1], semi).wait()

                pltpu.async_copy(x_hbm.at[src_b.at[m1, j1]], rows_v.at[q],
                                 semg)

            pltpu.make_async_copy(x_hbm.at[src_b.at[m, j]], rows_v.at[p],
                                  semg).wait()
            pltpu.sync_copy(rows_v.at[p], agg_sh.at[dst_b.at[m, j]], add=True)
            if with_deg:
                pltpu.sync_copy(ones_v, deg_sh.at[dst_b.at[m, j]], add=True)

            @pl.when(jnp.logical_and(j1 == 0, lb1 + 1 < NBLK))
            def _():
                pltpu.async_copy(src_hbm.at[wid, pl.ds((lb1 + 1) * NBC, NBC)],
                                 src_b.at[1 - m1], semi)
                pltpu.async_copy(dst_hbm.at[wid, pl.ds((lb1 + 1) * NBC, NBC)],
                                 dst_b.at[1 - m1], semi)

        plsc.subcore_barrier()
        pltpu.sync_copy(agg_sh.at[pl.ds(s * RPS, RPS)],
                        agg_out.at[c, pl.ds(s * RPS, RPS)])
        if with_deg:
            pltpu.sync_copy(deg_sh.at[pl.ds(s * RPS, RPS)],
                            deg_out.at[c, pl.ds(s * RPS, RPS)])

    return sc_kernel


_sc_agg_deg = _make_sc_agg(with_deg=True)
_sc_agg = _make_sc_agg(with_deg=False)

_BR = 2000


def _tc_layer1_body(aggp, degp, x, wl, bl, wr, h):
    agg = aggp[0] + aggp[1]
    deg = degp[0] + degp[1]
    mean = agg / jnp.maximum(deg[:, 0:1], 1.0)
    acc = lax.dot_general(mean, wl[...], (((1,), (1,)), ((), ())),
                          preferred_element_type=jnp.float32)
    acc += lax.dot_general(x[...], wr[...], (((1,), (1,)), ((), ())),
                           preferred_element_type=jnp.float32)
    h[...] = jnp.maximum(acc + bl[...], 0.0)


def _tc_layer2_body(aggp, degp, x, wl, bl, wr, out):
    agg = aggp[0] + aggp[1]
    deg = degp[0] + degp[1]
    mean = agg / jnp.maximum(deg[:, 0:1], 1.0)
    acc = lax.dot_general(mean, wl[...], (((1,), (1,)), ((), ())),
                          preferred_element_type=jnp.float32)
    acc += lax.dot_general(x[...], wr[...], (((1,), (1,)), ((), ())),
                           preferred_element_type=jnp.float32)
    o = acc + bl[...]
    m = jnp.max(o, axis=1, keepdims=True)
    lse = jnp.log(jnp.sum(jnp.exp(o - m), axis=1, keepdims=True)) + m
    out[...] = o - lse


def _tc_layer(body, aggp, degp, x, wl, bl, wr):
    def wrapped(aggp_ref, degp_ref, x_ref, wl_ref, bl_ref, wr_ref, o_ref):
        body(aggp_ref, degp_ref, x_ref, wl_ref, bl_ref, wr_ref, o_ref)

    return pl.pallas_call(
        wrapped,
        grid=(N // _BR,),
        in_specs=[
            pl.BlockSpec((NC, _BR, D), lambda i: (0, i, 0)),
            pl.BlockSpec((NC, _BR, 16), lambda i: (0, i, 0)),
            pl.BlockSpec((_BR, D), lambda i: (i, 0)),
            pl.BlockSpec((D, D), lambda i: (0, 0)),
            pl.BlockSpec((1, D), lambda i: (0, 0)),
            pl.BlockSpec((D, D), lambda i: (0, 0)),
        ],
        out_specs=pl.BlockSpec((_BR, D), lambda i: (i, 0)),
        out_shape=jax.ShapeDtypeStruct((N, D), jnp.float32),
    )(aggp, degp, x, wl, bl, wr)


def kernel(x, edge_index, Wl1, bl1, Wr1, Wl2, bl2, Wr2):
    src = edge_index[0].astype(jnp.int32).reshape(NW, NITER, CH)
    dst = edge_index[1].astype(jnp.int32).reshape(NW, NITER, CH)
    aggp1, degp = _sc_agg_deg(src, dst, x)
    h = _tc_layer(_tc_layer1_body, aggp1, degp, x, Wl1,
                  bl1.reshape(1, D), Wr1)
    aggp2, _ = _sc_agg_deg(src, dst, h)
    out = _tc_layer(_tc_layer2_body, aggp2, degp, h, Wl2,
                    bl2.reshape(1, D), Wr2)
    return out


# speedup vs baseline: 1.0310x; 1.0310x over previous
import functools

import jax
import jax.numpy as jnp
from jax import lax
from jax.experimental import pallas as pl
from jax.experimental.pallas import tpu as pltpu
from jax.experimental.pallas import tpu_sc as plsc

N = 10000
E = 320000
D = 128

NC = 2
NS = 16
NW = NC * NS
CH = 80
EPW = E // NW
NITER = EPW // CH
NBC = 25
NBLK = NITER // NBC
N_PAD = 10240
RPS = N_PAD // NS
ZB = 32


def _make_sc_agg(with_deg: bool):
    mesh = plsc.VectorSubcoreMesh(core_axis_name="c", subcore_axis_name="s")
    out_type = [jax.ShapeDtypeStruct((NC, N_PAD, D), jnp.float32)]
    if with_deg:
        out_type.append(jax.ShapeDtypeStruct((NC, N_PAD, 16), jnp.float32))
    scratch = [
        pltpu.VMEM((2, NBC, CH), jnp.int32),
        pltpu.VMEM((2, NBC, CH), jnp.int32),
        pltpu.VMEM((2, CH, D), jnp.float32),
        pltpu.VMEM((ZB, D), jnp.float32),
        pltpu.VMEM_SHARED((N_PAD, D), jnp.float32),
        pltpu.SemaphoreType.DMA,
        pltpu.SemaphoreType.DMA,
        pltpu.SemaphoreType.DMA,
        pltpu.SemaphoreType.DMA,
        pltpu.SemaphoreType.DMA,
    ]
    if with_deg:
        scratch += [
            pltpu.VMEM((CH, 16), jnp.float32),
            pltpu.VMEM((ZB, 16), jnp.float32),
            pltpu.VMEM_SHARED((N_PAD, 16), jnp.float32),
        ]

    @functools.partial(pl.kernel, out_type=tuple(out_type), mesh=mesh,
                       scratch_types=scratch,
                       compiler_params=pltpu.CompilerParams(
                           use_tc_tiling_on_sc=False))
    def sc_kernel(src_hbm, dst_hbm, x_hbm, *refs):
        if with_deg:
            (agg_out, deg_out, src_b, dst_b, rows_v, zrow_v, agg_sh,
             semg, semi, semz, sema, semd, ones_v, zdeg_v, deg_sh) = refs
        else:
            (agg_out, src_b, dst_b, rows_v, zrow_v, agg_sh,
             semg, semi, semz, sema, semd) = refs
        c = lax.axis_index("c")
        s = lax.axis_index("s")
        wid = s * NC + c
        zero16 = jnp.zeros((16,), jnp.float32)

        pltpu.sync_copy(src_hbm.at[wid, pl.ds(0, NBC)], src_b.at[0])
        pltpu.sync_copy(dst_hbm.at[wid, pl.ds(0, NBC)], dst_b.at[0])
        pltpu.async_copy(src_hbm.at[wid, pl.ds(NBC, NBC)], src_b.at[1], semi)
        pltpu.async_copy(dst_hbm.at[wid, pl.ds(NBC, NBC)], dst_b.at[1], semi)

        @pl.loop(0, ZB)
        def _(i):
            @pl.loop(0, D, step=16)
            def _(j):
                zrow_v[i, pl.ds(j, 16)] = zero16

        if with_deg:
            @pl.loop(0, ZB)
            def _(i):
                zdeg_v[i, pl.ds(0, 16)] = zero16

            @pl.loop(0, CH)
            def _(i):
                ones_v[i, pl.ds(0, 16)] = jnp.ones((16,), jnp.float32)

        @pl.loop(0, RPS, step=ZB)
        def _(k):
            pltpu.async_copy(zrow_v, agg_sh.at[pl.ds(s * RPS + k, ZB)], semz)
        if with_deg:
            @pl.loop(0, RPS, step=ZB)
            def _(k):
                pltpu.async_copy(zdeg_v, deg_sh.at[pl.ds(s * RPS + k, ZB)],
                                 semz)

        @pl.loop(0, RPS, step=ZB)
        def _(k):
            pltpu.make_async_copy(
                zrow_v, agg_sh.at[pl.ds(s * RPS + k, ZB)], semz).wait()
        if with_deg:
            @pl.loop(0, RPS, step=ZB)
            def _(k):
                pltpu.make_async_copy(
                    zdeg_v, deg_sh.at[pl.ds(s * RPS + k, ZB)], semz).wait()
        plsc.subcore_barrier()

        pltpu.async_copy(x_hbm.at[src_b.at[0, 0]], rows_v.at[0], semg)

        @pl.loop(0, NITER)
        def _(i):
            p = lax.rem(i, 2)
            q = 1 - p
            lb = lax.div(i, NBC)
            j = lax.rem(i, NBC)
            m = lax.rem(lb, 2)
            i1 = i + 1
            lb1 = lax.div(i1, NBC)
            j1 = lax.rem(i1, NBC)
            m1 = lax.rem(lb1, 2)

            @pl.when(i >= 1)
            def _():
                i0 = i - 1
                p0 = lax.rem(i0, 2)
                m0 = lax.rem(lax.div(i0, NBC), 2)
                j0 = lax.rem(i0, NBC)
                pltpu.make_async_copy(
                    rows_v.at[p0], agg_sh.at[dst_b.at[m0, j0]], sema).wait()
                if with_deg:
                    pltpu.make_async_copy(
                        ones_v, deg_sh.at[dst_b.at[m0, j0]], semd).wait()

                @pl.when(jnp.logical_and(j == 0, lb + 1 < NBLK))
                def _():
                    pltpu.async_copy(
                        src_hbm.at[wid, pl.ds((lb + 1) * NBC, NBC)],
                        src_b.at[1 - m], semi)
                    pltpu.async_copy(
                        dst_hbm.at[wid, pl.ds((lb + 1) * NBC, NBC)],
                        dst_b.at[1 - m], semi)

            @pl.when(i1 < NITER)
            def _():
                @pl.when(j1 == 0)
                def _():
                    pltpu.make_async_copy(
                        src_hbm.at[wid, pl.ds(lb1 * NBC, NBC)],
                        src_b.at[m1], semi).wait()
                    pltpu.make_async_copy(
                        dst_hbm.at[wid, pl.ds(lb1 * NBC, NBC)],
                        dst_b.at[m1], semi).wait()

                pltpu.async_copy(x_hbm.at[src_b.at[m1, j1]], rows_v.at[q],
                                 semg)

            pltpu.make_async_copy(x_hbm.at[src_b.at[m, j]], rows_v.at[p],
                                  semg).wait()
            pltpu.async_copy(rows_v.at[p], agg_sh.at[dst_b.at[m, j]], sema)
            if with_deg:
                pltpu.async_copy(ones_v, deg_sh.at[dst_b.at[m, j]], semd)

        _pL = (NITER - 1) % 2
        _mL = ((NITER - 1) // NBC) % 2
        _jL = (NITER - 1) % NBC
        pltpu.make_async_copy(
            rows_v.at[_pL], agg_sh.at[dst_b.at[_mL, _jL]], sema).wait()
        if with_deg:
            pltpu.make_async_copy(
                ones_v, deg_sh.at[dst_b.at[_mL, _jL]], semd).wait()

        plsc.subcore_barrier()
        pltpu.sync_copy(agg_sh.at[pl.ds(s * RPS, RPS)],
                        agg_out.at[c, pl.ds(s * RPS, RPS)])
        if with_deg:
            pltpu.sync_copy(deg_sh.at[pl.ds(s * RPS, RPS)],
                            deg_out.at[c, pl.ds(s * RPS, RPS)])

    return sc_kernel


_sc_agg_deg = _make_sc_agg(with_deg=True)
_sc_agg = _make_sc_agg(with_deg=False)

_BR = 2000


def _tc_layer1_body(aggp, degp, x, wl, bl, wr, h):
    agg = aggp[0] + aggp[1]
    deg = degp[0] + degp[1]
    mean = agg / jnp.maximum(deg[:, 0:1], 1.0)
    acc = lax.dot_general(mean, wl[...], (((1,), (1,)), ((), ())),
                          preferred_element_type=jnp.float32)
    acc += lax.dot_general(x[...], wr[...], (((1,), (1,)), ((), ())),
                           preferred_element_type=jnp.float32)
    h[...] = jnp.maximum(acc + bl[...], 0.0)


def _tc_layer2_body(aggp, degp, x, wl, bl, wr, out):
    agg = aggp[0] + aggp[1]
    deg = degp[0] + degp[1]
    mean = agg / jnp.maximum(deg[:, 0:1], 1.0)
    acc = lax.dot_general(mean, wl[...], (((1,), (1,)), ((), ())),
                          preferred_element_type=jnp.float32)
    acc += lax.dot_general(x[...], wr[...], (((1,), (1,)), ((), ())),
                           preferred_element_type=jnp.float32)
    o = acc + bl[...]
    m = jnp.max(o, axis=1, keepdims=True)
    lse = jnp.log(jnp.sum(jnp.exp(o - m), axis=1, keepdims=True)) + m
    out[...] = o - lse


def _tc_layer(body, aggp, degp, x, wl, bl, wr):
    def wrapped(aggp_ref, degp_ref, x_ref, wl_ref, bl_ref, wr_ref, o_ref):
        body(aggp_ref, degp_ref, x_ref, wl_ref, bl_ref, wr_ref, o_ref)

    return pl.pallas_call(
        wrapped,
        grid=(N // _BR,),
        in_specs=[
            pl.BlockSpec((NC, _BR, D), lambda i: (0, i, 0)),
            pl.BlockSpec((NC, _BR, 16), lambda i: (0, i, 0)),
            pl.BlockSpec((_BR, D), lambda i: (i, 0)),
            pl.BlockSpec((D, D), lambda i: (0, 0)),
            pl.BlockSpec((1, D), lambda i: (0, 0)),
            pl.BlockSpec((D, D), lambda i: (0, 0)),
        ],
        out_specs=pl.BlockSpec((_BR, D), lambda i: (i, 0)),
        out_shape=jax.ShapeDtypeStruct((N, D), jnp.float32),
    )(aggp, degp, x, wl, bl, wr)


def kernel(x, edge_index, Wl1, bl1, Wr1, Wl2, bl2, Wr2):
    src = edge_index[0].astype(jnp.int32).reshape(NW, NITER, CH)
    dst = edge_index[1].astype(jnp.int32).reshape(NW, NITER, CH)
    aggp1, degp = _sc_agg_deg(src, dst, x)
    h = _tc_layer(_tc_layer1_body, aggp1, degp, x, Wl1,
                  bl1.reshape(1, D), Wr1)
    aggp2, _ = _sc_agg_deg(src, dst, h)
    out = _tc_layer(_tc_layer2_body, aggp2, degp, h, Wl2,
                    bl2.reshape(1, D), Wr2)
    return out
